# baseline (speedup 1.0000x reference)
import jax
import jax.numpy as jnp
from jax import lax
from jax.experimental import pallas as pl
from jax.experimental.pallas import tpu as pltpu
from jax.experimental.pallas import tpu_sc as plsc

NUM_RELATIONS = 1000
DIM = 128
BATCH = 4096
L = 16
NW = 32
BPW = BATCH // NW
CH = 2
NCHUNK = BPW // CH
NBLK = DIM // L


def _compute_item(ht_v, mat_v, out_v, ii, i):
    t_vecs = [ht_v[i, pl.ds(DIM + L * j, L)] for j in range(NBLK)]

    def blk_body(db, accs):
        hvec = ht_v[i, pl.ds(db * L, L)]
        accs = list(accs)
        for k in range(L):
            row = db * L + k
            c = [mat_v[ii, row, pl.ds(L * j, L)] * t_vecs[j] for j in range(4)]
            for j in range(4, NBLK):
                c[j % 4] = c[j % 4] + mat_v[ii, row, pl.ds(L * j, L)] * t_vecs[j]
            d = (c[0] + c[1]) + (c[2] + c[3])
            accs[k % 4] = accs[k % 4] + hvec[k] * d
        return tuple(accs)

    zero = jnp.zeros((L,), jnp.float32)
    accs = lax.fori_loop(0, NBLK, blk_body, (zero,) * 4)
    out_v[i] = (accs[0] + accs[1]) + (accs[2] + accs[3])


def _sc_body(ht_hbm, rel_hbm, table_hbm, out_hbm,
             idx_v, ht_v, mat0_v, mat1_v, out_v, sem0, sem1):
    cid = lax.axis_index("c")
    sid = lax.axis_index("s")
    wid = sid * 2 + cid
    base = wid * BPW

    pltpu.sync_copy(rel_hbm.at[wid], idx_v)
    pltpu.async_copy(table_hbm.at[idx_v.at[0]], mat0_v, sem0)
    pltpu.async_copy(table_hbm.at[idx_v.at[1]], mat1_v, sem1)
    pltpu.sync_copy(ht_hbm.at[pl.ds(base, BPW)], ht_v)

    def pair_body(p, _):
        c0 = 2 * p
        pltpu.make_async_copy(table_hbm.at[idx_v.at[c0]], mat0_v, sem0).wait()
        for ii in range(CH):
            _compute_item(ht_v, mat0_v, out_v, ii, c0 * CH + ii)
        pltpu.async_copy(table_hbm.at[idx_v.at[c0 + 2]], mat0_v, sem0)

        pltpu.make_async_copy(table_hbm.at[idx_v.at[c0 + 1]], mat1_v, sem1).wait()
        for ii in range(CH):
            _compute_item(ht_v, mat1_v, out_v, ii, (c0 + 1) * CH + ii)
        pltpu.async_copy(table_hbm.at[idx_v.at[c0 + 3]], mat1_v, sem1)
        return 0

    lax.fori_loop(0, NCHUNK // 2, pair_body, 0)

    pltpu.make_async_copy(table_hbm.at[idx_v.at[NCHUNK]], mat0_v, sem0).wait()
    pltpu.make_async_copy(table_hbm.at[idx_v.at[NCHUNK + 1]], mat1_v, sem1).wait()

    pltpu.sync_copy(out_v, out_hbm.at[pl.ds(base, BPW)])


@jax.jit
def _bilinear_sc(ht, rel, table):
    mesh = plsc.VectorSubcoreMesh(core_axis_name="c", subcore_axis_name="s")
    fn = pl.kernel(
        _sc_body,
        out_type=jax.ShapeDtypeStruct((BATCH, L), jnp.float32),
        mesh=mesh,
        scratch_types=[
            pltpu.VMEM((NCHUNK + 2, CH), jnp.int32),
            pltpu.VMEM((BPW, 2 * DIM), jnp.float32),
            pltpu.VMEM((CH, DIM, DIM), jnp.float32),
            pltpu.VMEM((CH, DIM, DIM), jnp.float32),
            pltpu.VMEM((BPW, L), jnp.float32),
            pltpu.SemaphoreType.DMA,
            pltpu.SemaphoreType.DMA,
        ],
    )
    return fn(ht, rel, table)


def kernel(heads_and_tails, relations, kernel):
    rel = relations[:, 0].astype(jnp.int32).reshape(NW, NCHUNK, CH)
    rel = jnp.pad(rel, ((0, 0), (0, 2), (0, 0)))
    out16 = _bilinear_sc(heads_and_tails, rel, kernel)
    return jnp.sum(out16, axis=1)[:, None]


# speedup vs baseline: 1.2049x; 1.2049x over previous
import jax
import jax.numpy as jnp
from jax import lax
from jax.experimental import pallas as pl
from jax.experimental.pallas import tpu as pltpu
from jax.experimental.pallas import tpu_sc as plsc

NUM_RELATIONS = 1000
DIM = 128
BATCH = 4096
L = 16
NW = 32
BPW = BATCH // NW
CH = 2
NCHUNK = BPW // CH
NBLK = DIM // L


def _compute_item(ht_v, mat_v, out_v, ii, i):
    t_vecs = [ht_v[i, pl.ds(DIM + L * j, L)] for j in range(NBLK)]

    def blk_body(db, accs):
        hvec = ht_v[i, pl.ds(db * L, L)]
        ae, ao = accs
        for k in range(L):
            row = db * L + k
            p = mat_v[ii, row, pl.ds(0, L)] * t_vecs[0]
            q = mat_v[ii, row, pl.ds(L, L)] * t_vecs[1]
            for j in range(2, NBLK):
                mj = mat_v[ii, row, pl.ds(L * j, L)]
                if j % 2 == 0:
                    p = p + mj * t_vecs[j]
                else:
                    q = q + mj * t_vecs[j]
            d = p + q
            if k % 2 == 0:
                ae = ae + hvec[k] * d
            else:
                ao = ao + hvec[k] * d
        return (ae, ao)

    zero = jnp.zeros((L,), jnp.float32)
    ae, ao = lax.fori_loop(0, NBLK, blk_body, (zero, zero))
    out_v[i] = ae + ao


def _sc_body(ht_hbm, rel_hbm, table_hbm, out_hbm,
             idx_v, ht_v, mat0_v, mat1_v, out_v, sem0, sem1):
    cid = lax.axis_index("c")
    sid = lax.axis_index("s")
    wid = sid * 2 + cid
    base = wid * BPW

    pltpu.sync_copy(rel_hbm.at[wid], idx_v)
    pltpu.async_copy(table_hbm.at[idx_v.at[0]], mat0_v, sem0)
    pltpu.async_copy(table_hbm.at[idx_v.at[1]], mat1_v, sem1)
    pltpu.sync_copy(ht_hbm.at[pl.ds(base, BPW)], ht_v)

    def pair_body(p, _):
        c0 = 2 * p
        pltpu.make_async_copy(table_hbm.at[idx_v.at[c0]], mat0_v, sem0).wait()
        for ii in range(CH):
            _compute_item(ht_v, mat0_v, out_v, ii, c0 * CH + ii)
        pltpu.async_copy(table_hbm.at[idx_v.at[c0 + 2]], mat0_v, sem0)

        pltpu.make_async_copy(table_hbm.at[idx_v.at[c0 + 1]], mat1_v, sem1).wait()
        for ii in range(CH):
            _compute_item(ht_v, mat1_v, out_v, ii, (c0 + 1) * CH + ii)
        pltpu.async_copy(table_hbm.at[idx_v.at[c0 + 3]], mat1_v, sem1)
        return 0

    lax.fori_loop(0, NCHUNK // 2, pair_body, 0)

    pltpu.make_async_copy(table_hbm.at[idx_v.at[NCHUNK]], mat0_v, sem0).wait()
    pltpu.make_async_copy(table_hbm.at[idx_v.at[NCHUNK + 1]], mat1_v, sem1).wait()

    pltpu.sync_copy(out_v, out_hbm.at[pl.ds(base, BPW)])


@jax.jit
def _bilinear_sc(ht, rel, table):
    mesh = plsc.VectorSubcoreMesh(core_axis_name="c", subcore_axis_name="s")
    fn = pl.kernel(
        _sc_body,
        out_type=jax.ShapeDtypeStruct((BATCH, L), jnp.float32),
        mesh=mesh,
        scratch_types=[
            pltpu.VMEM((NCHUNK + 2, CH), jnp.int32),
            pltpu.VMEM((BPW, 2 * DIM), jnp.float32),
            pltpu.VMEM((CH, DIM, DIM), jnp.float32),
            pltpu.VMEM((CH, DIM, DIM), jnp.float32),
            pltpu.VMEM((BPW, L), jnp.float32),
            pltpu.SemaphoreType.DMA,
            pltpu.SemaphoreType.DMA,
        ],
    )
    return fn(ht, rel, table)


def kernel(heads_and_tails, relations, kernel):
    rel = relations[:, 0].astype(jnp.int32).reshape(NW, NCHUNK, CH)
    rel = jnp.pad(rel, ((0, 0), (0, 2), (0, 0)))
    out16 = _bilinear_sc(heads_and_tails, rel, kernel)
    return jnp.sum(out16, axis=1)[:, None]
